# baseline (speedup 1.0000x reference)
import jax
import jax.numpy as jnp
from jax import lax
from jax.experimental import pallas as pl
from jax.experimental.pallas import tpu as pltpu
from jax.experimental.pallas import tpu_sc as plsc

NUM_FIELDS = 26
VOCAB = 100000
DIM = 16
BATCH = 4096

NC = 2
NS = 16
L = 16
NW = NC * NS

ROWS = BATCH * NUM_FIELDS
ROWS_W = ROWS // NW


def _gather_body(idx_hbm, table_hbm, out_hbm, idx_v, rows_v, sem):
    wid = lax.axis_index("s") * NC + lax.axis_index("c")
    base = wid * ROWS_W
    pltpu.sync_copy(idx_hbm.at[pl.ds(base, ROWS_W)], idx_v)

    def add_off(j, carry):
        pos = base + j * L + lax.iota(jnp.int32, L)
        off = lax.rem(pos, NUM_FIELDS) * VOCAB
        idx_v[pl.ds(j * L, L)] = idx_v[pl.ds(j * L, L)] + off
        return carry

    lax.fori_loop(0, ROWS_W // L, add_off, 0, unroll=4)

    pltpu.async_copy(table_hbm.at[idx_v], rows_v, sem).wait()
    pltpu.sync_copy(rows_v, out_hbm.at[pl.ds(base, ROWS_W)])


@jax.jit
def _embed(indices_flat, table_flat):
    mesh = plsc.VectorSubcoreMesh(
        core_axis_name="c", subcore_axis_name="s", num_cores=NC, num_subcores=NS
    )
    return pl.kernel(
        _gather_body,
        out_type=jax.ShapeDtypeStruct((ROWS, DIM), jnp.float32),
        mesh=mesh,
        scratch_types=[
            pltpu.VMEM((ROWS_W,), jnp.int32),
            pltpu.VMEM((ROWS_W, DIM), jnp.float32),
            pltpu.SemaphoreType.DMA,
        ],
        compiler_params=pltpu.CompilerParams(use_tc_tiling_on_sc=False),
    )(indices_flat, table_flat)


def kernel(indices, tables):
    indices_flat = indices.reshape(ROWS)
    table_flat = tables.reshape(NUM_FIELDS * VOCAB, DIM)
    out = _embed(indices_flat, table_flat)
    return out.reshape(BATCH, NUM_FIELDS * DIM)


# speedup vs baseline: 6.2286x; 6.2286x over previous
import jax
import jax.numpy as jnp
from jax import lax
from jax.experimental import pallas as pl
from jax.experimental.pallas import tpu as pltpu
from jax.experimental.pallas import tpu_sc as plsc

NUM_FIELDS = 26
VOCAB = 100000
DIM = 16
BATCH = 4096

NC = 2
NS = 16
L = 16
NW = NC * NS

R = NUM_FIELDS * DIM
TR = R // 8
CHUNK0 = 51200
CHUNK1 = 48768
MAIN = CHUNK0 + CHUNK1
TAIL = VOCAB - MAIN
GROUPS = BATCH // L


def _row_body(idx_hbm, tab_hbm, tail_hbm, out_hbm, idx_v, chunk_v, tail_v, out_v):
    wid = lax.axis_index("s") * NC + lax.axis_index("c")
    ntr = jnp.where(wid < TR - NW, 2, 1)
    pltpu.sync_copy(tail_hbm, tail_v)

    def extract(lo, sz, first):
        def do_group(g, carry2):
            iv = idx_v[pl.ds(g * L, L)]
            m = (iv >= lo) & (iv < lo + sz)
            gv = plsc.load_gather(chunk_v.at[pl.ds(0, sz)], [iv - lo], mask=m)
            if first:
                out_v[pl.ds(g * L, L)] = gv
            else:
                prev = out_v[pl.ds(g * L, L)]
                out_v[pl.ds(g * L, L)] = jnp.where(m, gv, prev)
            return carry2

        lax.fori_loop(0, GROUPS, do_group, 0)

    def extract_tail(rbase):
        def do_group(g, carry2):
            iv = idx_v[pl.ds(g * L, L)]
            m = iv >= MAIN
            gv = plsc.load_gather(tail_v, [iv - MAIN + rbase], mask=m)
            prev = out_v[pl.ds(g * L, L)]
            out_v[pl.ds(g * L, L)] = jnp.where(m, gv, prev)
            return carry2

        lax.fori_loop(0, GROUPS, do_group, 0)

    def do_octet(k, carry):
        tr = wid + k * NW
        f = tr // 2
        pltpu.sync_copy(idx_hbm.at[f, :], idx_v)
        for d in range(8):
            for lo, sz, first in ((0, CHUNK0, True), (CHUNK0, CHUNK1, False)):
                pltpu.sync_copy(tab_hbm.at[tr, d, pl.ds(lo, sz)],
                                chunk_v.at[pl.ds(0, sz)])
                extract(lo, sz, first)
            extract_tail((tr * 8 + d) * TAIL)
            pltpu.sync_copy(out_v, out_hbm.at[tr, d, :])
        return carry

    lax.fori_loop(0, ntr, do_octet, 0)


@jax.jit
def _embed_t(idx_t, tab3, tail1):
    mesh = plsc.VectorSubcoreMesh(
        core_axis_name="c", subcore_axis_name="s", num_cores=NC, num_subcores=NS
    )
    return pl.kernel(
        _row_body,
        out_type=jax.ShapeDtypeStruct((TR, 8, BATCH), jnp.float32),
        mesh=mesh,
        scratch_types=[
            pltpu.VMEM((BATCH,), jnp.int32),
            pltpu.VMEM((CHUNK0,), jnp.float32),
            pltpu.VMEM((R * TAIL,), jnp.float32),
            pltpu.VMEM((BATCH,), jnp.float32),
        ],
        compiler_params=pltpu.CompilerParams(
            use_tc_tiling_on_sc=True, needs_layout_passes=False
        ),
    )(idx_t, tab3, tail1)


def kernel(indices, tables):
    idx_t = indices.T
    tab3 = jnp.transpose(tables, (0, 2, 1)).reshape(TR, 8, VOCAB)
    tail1 = jnp.transpose(tables[:, MAIN:, :], (0, 2, 1)).reshape(R * TAIL)
    out_t = _embed_t(idx_t, tab3, tail1)
    return out_t.reshape(R, BATCH).T


# speedup vs baseline: 9.0978x; 1.4606x over previous
import jax
import jax.numpy as jnp
from jax import lax
from jax.experimental import pallas as pl
from jax.experimental.pallas import tpu as pltpu
from jax.experimental.pallas import tpu_sc as plsc

NUM_FIELDS = 26
VOCAB = 100000
DIM = 16
BATCH = 4096

NC = 2
NS = 16
L = 16
NW = NC * NS

R = NUM_FIELDS * DIM
TR = R // 8
R_W = R // NW
MAIN = 99968
TAIL = VOCAB - MAIN
GROUPS = BATCH // L


def _row_body(idx_hbm, tab_hbm, tail_hbm, out_hbm, idx_v, row_v, tail_v, out_v):
    wid = lax.axis_index("s") * NC + lax.axis_index("c")
    lo_row = wid * R_W
    pltpu.sync_copy(tail_hbm, tail_v)

    def extract(rbase):
        def do_group(g, carry2):
            iv = idx_v[pl.ds(g * L, L)]
            m = iv < MAIN
            gv = plsc.load_gather(row_v, [iv], mask=m)
            tv = plsc.load_gather(tail_v, [iv - MAIN + rbase], mask=~m)
            out_v[pl.ds(g * L, L)] = jnp.where(m, gv, tv)
            return carry2

        lax.fori_loop(0, GROUPS, do_group, 0)

    for d in range(8):
        t_lo = (lo_row + 7 - d) // 8
        t_hi = (lo_row + R_W + 7 - d) // 8

        def do_row(t, carry, d=d):
            r = t * 8 + d
            f = r // DIM
            pltpu.sync_copy(idx_hbm.at[f, :], idx_v)
            pltpu.sync_copy(tab_hbm.at[t, d, pl.ds(0, MAIN)], row_v)
            extract(r * TAIL)
            pltpu.sync_copy(out_v, out_hbm.at[t, d, :])
            return carry

        lax.fori_loop(t_lo, t_hi, do_row, 0)


@jax.jit
def _embed_t(idx_t, tab3, tail1):
    mesh = plsc.VectorSubcoreMesh(
        core_axis_name="c", subcore_axis_name="s", num_cores=NC, num_subcores=NS
    )
    return pl.kernel(
        _row_body,
        out_type=jax.ShapeDtypeStruct((TR, 8, BATCH), jnp.float32),
        mesh=mesh,
        scratch_types=[
            pltpu.VMEM((BATCH,), jnp.int32),
            pltpu.VMEM((MAIN,), jnp.float32),
            pltpu.VMEM((R * TAIL,), jnp.float32),
            pltpu.VMEM((BATCH,), jnp.float32),
        ],
        compiler_params=pltpu.CompilerParams(
            use_tc_tiling_on_sc=True, needs_layout_passes=False
        ),
    )(idx_t, tab3, tail1)


def kernel(indices, tables):
    idx_t = indices.T
    tab3 = jnp.transpose(tables, (0, 2, 1)).reshape(TR, 8, VOCAB)
    tail1 = jnp.transpose(tables[:, MAIN:, :], (0, 2, 1)).reshape(R * TAIL)
    out_t = _embed_t(idx_t, tab3, tail1)
    return out_t.reshape(R, BATCH).T
